# baseline (speedup 1.0000x reference)
import functools

import jax
import jax.numpy as jnp
from jax import lax
from jax.experimental import pallas as pl
from jax.experimental.pallas import tpu as pltpu
from jax.experimental.pallas import tpu_sc as plsc

NC, NS = 2, 16
NW = NC * NS
BCHUNK = 128
RING = 4
GLAG = 2


def _emb_body(idx_hbm, table_hbm, out_hbm, idx_v, g_ring, t_ring, gsem, wsem,
              *, hist, b_per_w):
    wid = lax.axis_index("s") * NC + lax.axis_index("c")
    b0 = wid * b_per_w
    n_c = b_per_w // BCHUNK
    n_steps = hist * n_c

    pltpu.sync_copy(idx_hbm.at[:, pl.ds(b0, b_per_w)], idx_v)

    rows = [lax.iota(jnp.int32, 16) + 16 * i for i in range(BCHUNK // 16)]

    def _fire_gather(step, slot):
        h = step // n_c
        c = lax.rem(step, n_c)
        pltpu.async_copy(
            table_hbm.at[idx_v.at[h, pl.ds(c * BCHUNK, BCHUNK)]],
            g_ring[slot], gsem.at[slot],
        )

    def _fire_write(step, slot):
        h = step // n_c
        c = lax.rem(step, n_c)
        pltpu.async_copy(
            t_ring[slot],
            out_hbm.at[h, :, pl.ds(b0 + c * BCHUNK, BCHUNK)],
            wsem.at[slot],
        )

    def _wait_gather(slot):
        pltpu.make_async_copy(
            table_hbm.at[pl.ds(0, BCHUNK)], g_ring[slot], gsem.at[slot]
        ).wait()

    def _wait_write(slot):
        pltpu.make_async_copy(
            t_ring[slot], out_hbm.at[0, :, pl.ds(0, BCHUNK)], wsem.at[slot]
        ).wait()

    def _transpose(gbuf, tbuf):
        @pl.loop(0, 64)
        def _(d):
            col = jnp.full((16,), d, jnp.int32)
            for i in range(BCHUNK // 16):
                v = plsc.load_gather(gbuf, [rows[i], col])
                tbuf[d, pl.ds(16 * i, 16)] = v

    for s in range(GLAG):
        _fire_gather(s, s)

    @pl.loop(0, n_steps, step=RING)
    def _(s0):
        for b in range(RING):
            slot = b

            @pl.when(s0 + b >= RING)
            def _():
                _wait_write(slot)

            nxt = s0 + b + GLAG

            @pl.when(nxt < n_steps)
            def _():
                _fire_gather(nxt, (b + GLAG) % RING)

            _wait_gather(slot)
            _transpose(g_ring[slot], t_ring[slot])
            _fire_write(s0 + b, slot)

    for s in range(RING):
        _wait_write(s)


def kernel(inputs, weight):
    bsz, hist = inputs.shape
    vocab, dim = weight.shape
    assert bsz % (NW * BCHUNK) == 0 and dim == 64
    b_per_w = bsz // NW

    idx_t = jnp.transpose(inputs).astype(jnp.int32)

    run = pl.kernel(
        functools.partial(_emb_body, hist=hist, b_per_w=b_per_w),
        out_type=jax.ShapeDtypeStruct((hist, dim, bsz), jnp.float32),
        mesh=plsc.VectorSubcoreMesh(
            core_axis_name="c", subcore_axis_name="s",
            num_cores=NC, num_subcores=NS,
        ),
        scratch_types=[
            pltpu.VMEM((hist, b_per_w), jnp.int32),
            [pltpu.VMEM((BCHUNK, dim), jnp.float32) for _ in range(RING)],
            [pltpu.VMEM((dim, BCHUNK), jnp.float32) for _ in range(RING)],
            pltpu.SemaphoreType.DMA((RING,)),
            pltpu.SemaphoreType.DMA((RING,)),
        ],
        compiler_params=pltpu.CompilerParams(
            use_tc_tiling_on_sc=False, needs_layout_passes=False
        ),
    )
    raw = run(idx_t, weight)
    return jnp.transpose(raw, (2, 0, 1))
